# baseline (speedup 1.0000x reference)
import jax
import jax.numpy as jnp
from jax.experimental import pallas as pl

_B, _H, _W = 32, 512, 512
_BB = 2


def _loss_kernel(xw_ref, xs_ref, pm_ref, nm_ref, num_ref, cnt_ref, sxw_ref):
    i = pl.program_id(0)

    @pl.when(i == 0)
    def _init():
        num_ref[:, :] = jnp.zeros((1, 1), jnp.float32)
        cnt_ref[:, :] = jnp.zeros((1, 1), jnp.float32)

    xw = xw_ref[...]
    xs = xs_ref[...]
    mask = (pm_ref[...] > 0.5) & (nm_ref[...] > 0.5)

    sp_w = jnp.maximum(-xw, 0.0) + jnp.log1p(jnp.exp(-jnp.abs(xw)))
    sp_s = jnp.maximum(-xs, 0.0) + jnp.log1p(jnp.exp(-jnp.abs(xs)))

    pseudo = jnp.where((xw > 0.6) | (xw < 0.4), xw, 0.0)
    t_sum = (sp_w + sp_s) + (xw + xs) - pseudo * xs

    num_ref[:, :] += jnp.sum(jnp.where(mask, t_sum, 0.0)).reshape(1, 1)
    cnt_ref[:, :] += jnp.sum(jnp.where(mask, 1.0, 0.0)).reshape(1, 1)
    sxw_ref[:, :, :] = jnp.sum(jnp.where(mask, xw, 0.0),
                               axis=(1, 2)).reshape(1, _BB, 1)


def kernel(logits_w, logits_s, prostate_mask, needle_mask, ood_mask,
           label, involvement):
    del ood_mask, involvement
    xw = logits_w.reshape(_B, _H, _W)
    xs = logits_s.reshape(_B, _H, _W)
    pm = prostate_mask.reshape(_B, _H, _W)
    nm = needle_mask.reshape(_B, _H, _W)

    blk = pl.BlockSpec((_BB, _H, _W), lambda i: (i, 0, 0))
    scal_blk = pl.BlockSpec((1, 1), lambda i: (0, 0))

    num, cnt, sxw = pl.pallas_call(
        _loss_kernel,
        grid=(_B // _BB,),
        in_specs=[blk, blk, blk, blk],
        out_specs=[scal_blk, scal_blk,
                   pl.BlockSpec((1, _BB, 1), lambda i: (i, 0, 0))],
        out_shape=[jax.ShapeDtypeStruct((1, 1), jnp.float32),
                   jax.ShapeDtypeStruct((1, 1), jnp.float32),
                   jax.ShapeDtypeStruct((_B // _BB, _BB, 1), jnp.float32)],
    )(xw, xs, pm, nm)

    lab_dot = jnp.dot(label.astype(jnp.float32), sxw.reshape(_B))
    return (0.5 * (num[0, 0] - lab_dot) / cnt[0, 0]).astype(jnp.float32)
